# baseline (speedup 1.0000x reference)
import functools

import jax
import jax.numpy as jnp
from jax import lax
from jax.experimental import pallas as pl
from jax.experimental.pallas import tpu as pltpu
from jax.experimental.pallas import tpu_sc as plsc

_NC = 2
_NS = 16
_NW = _NC * _NS
_C = 128
_NBUF = 4
_L = 16


@jax.jit
def _sc_transpose(tableT, tail):
    d, v = tableT.shape
    slabs = v // _C
    rem = v - slabs * _C
    n_visits = -(-slabs // _NW)
    mesh = plsc.VectorSubcoreMesh(core_axis_name="c", subcore_axis_name="s")

    scratch = (
        [pltpu.VMEM((d, _C), jnp.float32) for _ in range(2)]
        + [pltpu.VMEM((_C, _C), jnp.float32) for _ in range(2)]
        + [pltpu.VMEM((d, rem), jnp.float32)]
        + [pltpu.SemaphoreType.DMA for _ in range(4)]
    )

    @functools.partial(
        pl.kernel,
        mesh=mesh,
        out_type=jax.ShapeDtypeStruct((v, _C), jnp.float32),
        scratch_types=scratch,
        compiler_params=pltpu.CompilerParams(
            use_tc_tiling_on_sc=True, needs_layout_passes=False),
    )
    def k(tableT_hbm, tail_hbm, out_hbm, src0, src1, dst0, dst1, tailv,
          in0, in1, out0, out1):
        src = (src0, src1)
        dst = (dst0, dst1)
        in_sem = (in0, in1)
        out_sem = (out0, out1)
        wid = lax.axis_index("s") * _NC + lax.axis_index("c")
        rows16 = lax.iota(jnp.int32, _L)

        def c_eff(kv):
            c = wid + kv * _NW
            return jnp.where(c < slabs, c, 0)

        def start_in(kv, b):
            pltpu.async_copy(
                tableT_hbm.at[:, pl.ds(c_eff(kv) * _C, _C)], src[b], in_sem[b])

        def wait_in(b):
            pltpu.make_async_copy(
                tableT_hbm.at[:, pl.ds(0, _C)], src[b], in_sem[b]).wait()

        def start_out(kv, b):
            pltpu.async_copy(
                dst[b], out_hbm.at[pl.ds(c_eff(kv) * _C, _C)], out_sem[b])

        def wait_out(b):
            pltpu.make_async_copy(
                dst[b], out_hbm.at[pl.ds(0, _C)], out_sem[b]).wait()

        def transpose(s, t, ncols):
            def row_body(i, carry):
                for d0 in range(0, d, _L):
                    cols = jnp.full((_L,), i, jnp.int32)
                    vals = plsc.load_gather(s, [rows16 + d0, cols])
                    t[i, pl.ds(d0, _L)] = vals
                return carry
            lax.fori_loop(0, ncols, row_body, 0)

        start_in(0, 0)
        start_in(1, 1)
        start_out(0, 0)
        start_out(1, 1)

        def visit(kv, b):
            wait_in(b)
            wait_out(b)
            transpose(src[b], dst[b], _C)
            start_out(kv, b)
            start_in(kv + 2, b)

        def body(g, carry):
            visit(2 * g, 0)
            visit(2 * g + 1, 1)
            return carry

        lax.fori_loop(0, n_visits // 2, body, 0)

        kv = n_visits - 1
        wait_in(0)
        wait_out(0)
        transpose(src[0], dst[0], _C)
        start_out(kv, 0)
        wait_in(1)
        wait_out(1)
        wait_out(0)

        @pl.when(wid == 0)
        def _():
            pltpu.sync_copy(tail_hbm, tailv)
            transpose(tailv, dst0, rem)
            pltpu.sync_copy(dst0.at[pl.ds(0, rem), :],
                            out_hbm.at[pl.ds(slabs * _C, rem)])

    return k(tableT, tail)


@functools.partial(jax.jit, static_argnums=(2, 3))
def _sc_gather(ids3, tableP, n_per_w, n_chunks):
    d = tableP.shape[1]
    n = n_per_w * _NW
    n_groups = n_chunks // _NBUF
    mesh = plsc.VectorSubcoreMesh(core_axis_name="c", subcore_axis_name="s")

    scratch = (
        [pltpu.VMEM((n_chunks, _C), jnp.int32)]
        + [pltpu.VMEM((_C, d), jnp.float32) for _ in range(_NBUF)]
        + [pltpu.SemaphoreType.DMA for _ in range(2 * _NBUF)]
    )

    @functools.partial(
        pl.kernel,
        mesh=mesh,
        out_type=jax.ShapeDtypeStruct((n, d), jnp.float32),
        scratch_types=scratch,
        compiler_params=pltpu.CompilerParams(use_tc_tiling_on_sc=True),
    )
    def k(ids_hbm, table_hbm, out_hbm, idx_v, *rest):
        bufs = rest[:_NBUF]
        in_sem = rest[_NBUF:2 * _NBUF]
        out_sem = rest[2 * _NBUF:]
        wid = lax.axis_index("s") * _NC + lax.axis_index("c")
        base = wid * n_per_w
        pltpu.sync_copy(ids_hbm.at[wid], idx_v)

        def start_gather(jn, b):
            pltpu.async_copy(table_hbm.at[idx_v.at[jn]], bufs[b], in_sem[b])

        def wait_gather(b):
            pltpu.make_async_copy(
                table_hbm.at[idx_v.at[0]], bufs[b], in_sem[b]).wait()

        def start_scatter(j, b):
            pltpu.async_copy(
                bufs[b], out_hbm.at[pl.ds(base + j * _C, _C)], out_sem[b])

        def wait_scatter(b):
            pltpu.make_async_copy(
                bufs[b], out_hbm.at[pl.ds(base, _C)], out_sem[b]).wait()

        for b in range(_NBUF - 1):
            start_gather(b, b)
        start_scatter(_NBUF - 1, _NBUF - 1)

        def visit(j, b):
            b1 = (b - 1) % _NBUF
            wait_scatter(b1)
            start_gather(j + _NBUF - 1, b1)
            wait_gather(b)
            start_scatter(j, b)

        def body(g, carry):
            for b in range(_NBUF):
                visit(g * _NBUF + b, b)
            return carry

        lax.fori_loop(0, n_groups - 1, body, 0)

        g0 = (n_groups - 1) * _NBUF
        for b in range(_NBUF):
            b1 = (b - 1) % _NBUF
            if b == 0:
                wait_scatter(b1)
                start_gather(g0 + _NBUF - 1, b1)
            wait_gather(b)
            start_scatter(g0 + b, b)
        for b in range(_NBUF):
            wait_scatter(b)

    return k(ids3, tableP)


def kernel(input_ids, table):
    b, l = input_ids.shape
    n = b * l
    v, d = table.shape
    assert n % (_NW * _C * _NBUF) == 0
    n_per_w = n // _NW
    n_chunks = n_per_w // _C
    ids3 = input_ids.reshape(_NW, n_chunks, _C)
    tableT = table.T
    rem = v % _C
    tail = tableT[:, v - rem:]
    tableP = _sc_transpose(tableT, tail)
    outP = _sc_gather(ids3, tableP, n_per_w, n_chunks)
    out = lax.slice(outP, (0, 0), (n, d))
    return out.reshape(b, l, d)


# speedup vs baseline: 1.1648x; 1.1648x over previous
import functools

import jax
import jax.numpy as jnp
from jax import lax
from jax.experimental import pallas as pl
from jax.experimental.pallas import tpu as pltpu
from jax.experimental.pallas import tpu_sc as plsc

_NC = 2
_NS = 16
_NW = _NC * _NS
_C = 128
_NBUF = 4
_L = 16


@jax.jit
def _sc_transpose(tableT, tail):
    d, v = tableT.shape
    slabs = v // _C
    rem = v - slabs * _C
    n_visits = -(-slabs // _NW)
    mesh = plsc.VectorSubcoreMesh(core_axis_name="c", subcore_axis_name="s")

    scratch = (
        [pltpu.VMEM((d, _C), jnp.float32) for _ in range(2)]
        + [pltpu.VMEM((_C, _C), jnp.float32) for _ in range(2)]
        + [pltpu.VMEM((d, rem), jnp.float32)]
        + [pltpu.SemaphoreType.DMA for _ in range(4)]
    )

    @functools.partial(
        pl.kernel,
        mesh=mesh,
        out_type=jax.ShapeDtypeStruct((v, _C), jnp.float32),
        scratch_types=scratch,
        compiler_params=pltpu.CompilerParams(
            use_tc_tiling_on_sc=True, needs_layout_passes=False),
    )
    def k(tableT_hbm, tail_hbm, out_hbm, src0, src1, dst0, dst1, tailv,
          in0, in1, out0, out1):
        src = (src0, src1)
        dst = (dst0, dst1)
        in_sem = (in0, in1)
        out_sem = (out0, out1)
        wid = lax.axis_index("s") * _NC + lax.axis_index("c")
        rows16 = lax.iota(jnp.int32, _L)

        def c_eff(kv):
            c = wid + kv * _NW
            return jnp.where(c < slabs, c, 0)

        def start_in(kv, b):
            pltpu.async_copy(
                tableT_hbm.at[:, pl.ds(c_eff(kv) * _C, _C)], src[b], in_sem[b])

        def wait_in(b):
            pltpu.make_async_copy(
                tableT_hbm.at[:, pl.ds(0, _C)], src[b], in_sem[b]).wait()

        def start_out(kv, b):
            pltpu.async_copy(
                dst[b], out_hbm.at[pl.ds(c_eff(kv) * _C, _C)], out_sem[b])

        def wait_out(b):
            pltpu.make_async_copy(
                dst[b], out_hbm.at[pl.ds(0, _C)], out_sem[b]).wait()

        def transpose(s, t, ncols):
            row_idx = [rows16 + c0 for c0 in range(0, ncols, _L)]

            def row_body(g, carry):
                for u in range(4):
                    dd = g * 4 + u
                    cols = jnp.full((_L,), dd, jnp.int32)
                    for ci, c0 in enumerate(range(0, ncols, _L)):
                        vals = s[dd, pl.ds(c0, _L)]
                        plsc.store_scatter(t, [row_idx[ci], cols], vals)
                return carry
            lax.fori_loop(0, d // 4, row_body, 0)

        start_in(0, 0)
        start_in(1, 1)
        start_out(0, 0)
        start_out(1, 1)

        def visit(kv, b):
            wait_in(b)
            wait_out(b)
            transpose(src[b], dst[b], _C)
            start_out(kv, b)
            start_in(kv + 2, b)

        def body(g, carry):
            visit(2 * g, 0)
            visit(2 * g + 1, 1)
            return carry

        lax.fori_loop(0, n_visits // 2, body, 0)

        kv = n_visits - 1
        wait_in(0)
        wait_out(0)
        transpose(src[0], dst[0], _C)
        start_out(kv, 0)
        wait_in(1)
        wait_out(1)
        wait_out(0)

        @pl.when(wid == 0)
        def _():
            pltpu.sync_copy(tail_hbm, tailv)
            transpose(tailv, dst0, rem)
            pltpu.sync_copy(dst0.at[pl.ds(0, rem), :],
                            out_hbm.at[pl.ds(slabs * _C, rem)])

    return k(tableT, tail)


@functools.partial(jax.jit, static_argnums=(2, 3))
def _sc_gather(ids3, tableP, n_per_w, n_chunks):
    d = tableP.shape[1]
    n = n_per_w * _NW
    n_groups = n_chunks // _NBUF
    mesh = plsc.VectorSubcoreMesh(core_axis_name="c", subcore_axis_name="s")

    scratch = (
        [pltpu.VMEM((n_chunks, _C), jnp.int32)]
        + [pltpu.VMEM((_C, d), jnp.float32) for _ in range(_NBUF)]
        + [pltpu.SemaphoreType.DMA for _ in range(2 * _NBUF)]
    )

    @functools.partial(
        pl.kernel,
        mesh=mesh,
        out_type=jax.ShapeDtypeStruct((n, d), jnp.float32),
        scratch_types=scratch,
        compiler_params=pltpu.CompilerParams(use_tc_tiling_on_sc=True),
    )
    def k(ids_hbm, table_hbm, out_hbm, idx_v, *rest):
        bufs = rest[:_NBUF]
        in_sem = rest[_NBUF:2 * _NBUF]
        out_sem = rest[2 * _NBUF:]
        wid = lax.axis_index("s") * _NC + lax.axis_index("c")
        base = wid * n_per_w
        pltpu.sync_copy(ids_hbm.at[wid], idx_v)

        def start_gather(jn, b):
            pltpu.async_copy(table_hbm.at[idx_v.at[jn]], bufs[b], in_sem[b])

        def wait_gather(b):
            pltpu.make_async_copy(
                table_hbm.at[idx_v.at[0]], bufs[b], in_sem[b]).wait()

        def start_scatter(j, b):
            pltpu.async_copy(
                bufs[b], out_hbm.at[pl.ds(base + j * _C, _C)], out_sem[b])

        def wait_scatter(b):
            pltpu.make_async_copy(
                bufs[b], out_hbm.at[pl.ds(base, _C)], out_sem[b]).wait()

        for b in range(_NBUF - 1):
            start_gather(b, b)
        start_scatter(_NBUF - 1, _NBUF - 1)

        def visit(j, b):
            b1 = (b - 1) % _NBUF
            wait_scatter(b1)
            start_gather(j + _NBUF - 1, b1)
            wait_gather(b)
            start_scatter(j, b)

        def body(g, carry):
            for b in range(_NBUF):
                visit(g * _NBUF + b, b)
            return carry

        lax.fori_loop(0, n_groups - 1, body, 0)

        g0 = (n_groups - 1) * _NBUF
        for b in range(_NBUF):
            b1 = (b - 1) % _NBUF
            if b == 0:
                wait_scatter(b1)
                start_gather(g0 + _NBUF - 1, b1)
            wait_gather(b)
            start_scatter(g0 + b, b)
        for b in range(_NBUF):
            wait_scatter(b)

    return k(ids3, tableP)


def kernel(input_ids, table):
    b, l = input_ids.shape
    n = b * l
    v, d = table.shape
    assert n % (_NW * _C * _NBUF) == 0
    n_per_w = n // _NW
    n_chunks = n_per_w // _C
    ids3 = input_ids.reshape(_NW, n_chunks, _C)
    tableT = table.T
    rem = v % _C
    tail = tableT[:, v - rem:]
    tableP = _sc_transpose(tableT, tail)
    outP = _sc_gather(ids3, tableP, n_per_w, n_chunks)
    out = lax.slice(outP, (0, 0), (n, d))
    return out.reshape(b, l, d)


# speedup vs baseline: 1.4588x; 1.2523x over previous
import functools

import jax
import jax.numpy as jnp
from jax import lax
from jax.experimental import pallas as pl
from jax.experimental.pallas import tpu as pltpu
from jax.experimental.pallas import tpu_sc as plsc

_NC = 2
_NS = 16
_NW = _NC * _NS
_C = 128
_NBUF = 4
_L = 16


@jax.jit
def _sc_transpose(tableT, tail):
    d, v = tableT.shape
    slabs = v // _C
    rem = v - slabs * _C
    n_visits = -(-slabs // _NW)
    mesh = plsc.VectorSubcoreMesh(core_axis_name="c", subcore_axis_name="s")

    scratch = (
        [pltpu.VMEM((d, _C), jnp.float32) for _ in range(2)]
        + [pltpu.VMEM((_C, _C), jnp.float32) for _ in range(2)]
        + [pltpu.VMEM((d, rem), jnp.float32)]
        + [pltpu.SemaphoreType.DMA for _ in range(4)]
    )

    @functools.partial(
        pl.kernel,
        mesh=mesh,
        out_type=jax.ShapeDtypeStruct((v, _C), jnp.float32),
        scratch_types=scratch,
        compiler_params=pltpu.CompilerParams(
            use_tc_tiling_on_sc=True, needs_layout_passes=False),
    )
    def k(tableT_hbm, tail_hbm, out_hbm, src0, src1, dst0, dst1, tailv,
          in0, in1, out0, out1):
        src = (src0, src1)
        dst = (dst0, dst1)
        in_sem = (in0, in1)
        out_sem = (out0, out1)
        wid = lax.axis_index("s") * _NC + lax.axis_index("c")
        rows16 = lax.iota(jnp.int32, _L)

        def c_eff(kv):
            c = wid + kv * _NW
            return jnp.where(c < slabs, c, 0)

        def start_in(kv, b):
            pltpu.async_copy(
                tableT_hbm.at[:, pl.ds(c_eff(kv) * _C, _C)], src[b], in_sem[b])

        def wait_in(b):
            pltpu.make_async_copy(
                tableT_hbm.at[:, pl.ds(0, _C)], src[b], in_sem[b]).wait()

        def start_out(kv, b):
            pltpu.async_copy(
                dst[b], out_hbm.at[pl.ds(c_eff(kv) * _C, _C)], out_sem[b])

        def wait_out(b):
            pltpu.make_async_copy(
                dst[b], out_hbm.at[pl.ds(0, _C)], out_sem[b]).wait()

        def transpose(s, t, ncols):
            row_idx = [rows16 + c0 for c0 in range(0, ncols, _L)]

            @plsc.parallel_loop(0, d // 4, unroll=2)
            def _(g):
                for u in range(4):
                    dd = g * 4 + u
                    cols = jnp.full((_L,), dd, jnp.int32)
                    for ci, c0 in enumerate(range(0, ncols, _L)):
                        vals = s[dd, pl.ds(c0, _L)]
                        plsc.store_scatter(t, [row_idx[ci], cols], vals)

        start_in(0, 0)
        start_in(1, 1)
        start_out(0, 0)
        start_out(1, 1)

        def visit(kv, b):
            wait_in(b)
            wait_out(b)
            transpose(src[b], dst[b], _C)
            start_out(kv, b)
            start_in(kv + 2, b)

        def body(g, carry):
            visit(2 * g, 0)
            visit(2 * g + 1, 1)
            return carry

        lax.fori_loop(0, n_visits // 2, body, 0)

        kv = n_visits - 1
        wait_in(0)
        wait_out(0)
        transpose(src[0], dst[0], _C)
        start_out(kv, 0)
        wait_in(1)
        wait_out(1)
        wait_out(0)

        @pl.when(wid == 0)
        def _():
            pltpu.sync_copy(tail_hbm, tailv)
            transpose(tailv, dst0, rem)
            pltpu.sync_copy(dst0.at[pl.ds(0, rem), :],
                            out_hbm.at[pl.ds(slabs * _C, rem)])

    return k(tableT, tail)


@functools.partial(jax.jit, static_argnums=(2, 3))
def _sc_gather(ids3, tableP, n_per_w, n_chunks):
    d = tableP.shape[1]
    n = n_per_w * _NW
    n_groups = n_chunks // _NBUF
    mesh = plsc.VectorSubcoreMesh(core_axis_name="c", subcore_axis_name="s")

    scratch = (
        [pltpu.VMEM((n_chunks, _C), jnp.int32)]
        + [pltpu.VMEM((_C, d), jnp.float32) for _ in range(_NBUF)]
        + [pltpu.SemaphoreType.DMA for _ in range(2 * _NBUF)]
    )

    @functools.partial(
        pl.kernel,
        mesh=mesh,
        out_type=jax.ShapeDtypeStruct((n, d), jnp.float32),
        scratch_types=scratch,
        compiler_params=pltpu.CompilerParams(use_tc_tiling_on_sc=True),
    )
    def k(ids_hbm, table_hbm, out_hbm, idx_v, *rest):
        bufs = rest[:_NBUF]
        in_sem = rest[_NBUF:2 * _NBUF]
        out_sem = rest[2 * _NBUF:]
        wid = lax.axis_index("s") * _NC + lax.axis_index("c")
        base = wid * n_per_w
        pltpu.sync_copy(ids_hbm.at[wid], idx_v)

        def start_gather(jn, b):
            pltpu.async_copy(table_hbm.at[idx_v.at[jn]], bufs[b], in_sem[b])

        def wait_gather(b):
            pltpu.make_async_copy(
                table_hbm.at[idx_v.at[0]], bufs[b], in_sem[b]).wait()

        def start_scatter(j, b):
            pltpu.async_copy(
                bufs[b], out_hbm.at[pl.ds(base + j * _C, _C)], out_sem[b])

        def wait_scatter(b):
            pltpu.make_async_copy(
                bufs[b], out_hbm.at[pl.ds(base, _C)], out_sem[b]).wait()

        for b in range(_NBUF - 1):
            start_gather(b, b)
        start_scatter(_NBUF - 1, _NBUF - 1)

        def visit(j, b):
            b1 = (b - 1) % _NBUF
            wait_scatter(b1)
            start_gather(j + _NBUF - 1, b1)
            wait_gather(b)
            start_scatter(j, b)

        def body(g, carry):
            for b in range(_NBUF):
                visit(g * _NBUF + b, b)
            return carry

        lax.fori_loop(0, n_groups - 1, body, 0)

        g0 = (n_groups - 1) * _NBUF
        for b in range(_NBUF):
            b1 = (b - 1) % _NBUF
            if b == 0:
                wait_scatter(b1)
                start_gather(g0 + _NBUF - 1, b1)
            wait_gather(b)
            start_scatter(g0 + b, b)
        for b in range(_NBUF):
            wait_scatter(b)

    return k(ids3, tableP)


def kernel(input_ids, table):
    b, l = input_ids.shape
    n = b * l
    v, d = table.shape
    assert n % (_NW * _C * _NBUF) == 0
    n_per_w = n // _NW
    n_chunks = n_per_w // _C
    ids3 = input_ids.reshape(_NW, n_chunks, _C)
    tableT = table.T
    rem = v % _C
    tail = tableT[:, v - rem:]
    tableP = _sc_transpose(tableT, tail)
    outP = _sc_gather(ids3, tableP, n_per_w, n_chunks)
    out = lax.slice(outP, (0, 0), (n, d))
    return out.reshape(b, l, d)


# speedup vs baseline: 2.5509x; 1.7487x over previous
import functools

import jax
import jax.numpy as jnp
from jax import lax
from jax.experimental import pallas as pl
from jax.experimental.pallas import tpu as pltpu
from jax.experimental.pallas import tpu_sc as plsc

_NC = 2
_NS = 16
_NW = _NC * _NS
_C = 128
_NBUF = 4
_L = 16


@jax.jit
def _sc_transpose(tableT, tail):
    d, v = tableT.shape
    slabs = v // _C
    rem = v - slabs * _C
    n_visits = -(-slabs // _NW)
    mesh = plsc.VectorSubcoreMesh(core_axis_name="c", subcore_axis_name="s")

    scratch = (
        [pltpu.VMEM((d, _C), jnp.float32) for _ in range(2)]
        + [pltpu.VMEM((_C, _C), jnp.float32) for _ in range(2)]
        + [pltpu.VMEM((d, rem), jnp.float32)]
        + [pltpu.SemaphoreType.DMA for _ in range(4)]
    )

    @functools.partial(
        pl.kernel,
        mesh=mesh,
        out_type=jax.ShapeDtypeStruct((v, _C), jnp.float32),
        scratch_types=scratch,
        compiler_params=pltpu.CompilerParams(
            use_tc_tiling_on_sc=True, needs_layout_passes=False),
    )
    def k(tableT_hbm, tail_hbm, out_hbm, src0, src1, dst0, dst1, tailv,
          in0, in1, out0, out1):
        src = (src0, src1)
        dst = (dst0, dst1)
        in_sem = (in0, in1)
        out_sem = (out0, out1)
        wid = lax.axis_index("s") * _NC + lax.axis_index("c")
        rows16 = lax.iota(jnp.int32, _L)

        def c_eff(kv):
            c = wid + kv * _NW
            return jnp.where(c < slabs, c, 0)

        def start_in(kv, b):
            pltpu.async_copy(
                tableT_hbm.at[:, pl.ds(c_eff(kv) * _C, _C)], src[b], in_sem[b])

        def wait_in(b):
            pltpu.make_async_copy(
                tableT_hbm.at[:, pl.ds(0, _C)], src[b], in_sem[b]).wait()

        def start_out(kv, b):
            pltpu.async_copy(
                dst[b], out_hbm.at[pl.ds(c_eff(kv) * _C, _C)], out_sem[b])

        def wait_out(b):
            pltpu.make_async_copy(
                dst[b], out_hbm.at[pl.ds(0, _C)], out_sem[b]).wait()

        diag = [jnp.bitwise_and(rows16 + k, _L - 1) for k in range(_L)]

        def transpose(s, t, ncols):
            nrb = d // _L

            @plsc.parallel_loop(0, nrb * (ncols // _L))
            def _(g):
                rr = lax.shift_right_logical(g, 3 if ncols == _C else 2)
                cb = lax.bitwise_and(g, (ncols // _L) - 1)
                rbase = rr * _L + rows16
                cbase = cb * _L
                for k in range(_L):
                    dk = cbase + diag[k]
                    vals = plsc.load_gather(s, [rbase, dk])
                    plsc.store_scatter(t, [dk, rbase], vals)

        start_in(0, 0)
        start_in(1, 1)
        start_out(0, 0)
        start_out(1, 1)

        def visit(kv, b):
            wait_in(b)
            wait_out(b)
            transpose(src[b], dst[b], _C)
            start_out(kv, b)
            start_in(kv + 2, b)

        def body(g, carry):
            visit(2 * g, 0)
            visit(2 * g + 1, 1)
            return carry

        lax.fori_loop(0, n_visits // 2, body, 0)

        kv = n_visits - 1
        wait_in(0)
        wait_out(0)
        transpose(src[0], dst[0], _C)
        start_out(kv, 0)
        wait_in(1)
        wait_out(1)
        wait_out(0)

        @pl.when(wid == 0)
        def _():
            pltpu.sync_copy(tail_hbm, tailv)
            transpose(tailv, dst0, rem)
            pltpu.sync_copy(dst0.at[pl.ds(0, rem), :],
                            out_hbm.at[pl.ds(slabs * _C, rem)])

    return k(tableT, tail)


@functools.partial(jax.jit, static_argnums=(2, 3))
def _sc_gather(ids3, tableP, n_per_w, n_chunks):
    d = tableP.shape[1]
    n = n_per_w * _NW
    n_groups = n_chunks // _NBUF
    mesh = plsc.VectorSubcoreMesh(core_axis_name="c", subcore_axis_name="s")

    scratch = (
        [pltpu.VMEM((n_chunks, _C), jnp.int32)]
        + [pltpu.VMEM((_C, d), jnp.float32) for _ in range(_NBUF)]
        + [pltpu.SemaphoreType.DMA for _ in range(2 * _NBUF)]
    )

    @functools.partial(
        pl.kernel,
        mesh=mesh,
        out_type=jax.ShapeDtypeStruct((n, d), jnp.float32),
        scratch_types=scratch,
        compiler_params=pltpu.CompilerParams(use_tc_tiling_on_sc=True),
    )
    def k(ids_hbm, table_hbm, out_hbm, idx_v, *rest):
        bufs = rest[:_NBUF]
        in_sem = rest[_NBUF:2 * _NBUF]
        out_sem = rest[2 * _NBUF:]
        wid = lax.axis_index("s") * _NC + lax.axis_index("c")
        base = wid * n_per_w
        pltpu.sync_copy(ids_hbm.at[wid], idx_v)

        def start_gather(jn, b):
            pltpu.async_copy(table_hbm.at[idx_v.at[jn]], bufs[b], in_sem[b])

        def wait_gather(b):
            pltpu.make_async_copy(
                table_hbm.at[idx_v.at[0]], bufs[b], in_sem[b]).wait()

        def start_scatter(j, b):
            pltpu.async_copy(
                bufs[b], out_hbm.at[pl.ds(base + j * _C, _C)], out_sem[b])

        def wait_scatter(b):
            pltpu.make_async_copy(
                bufs[b], out_hbm.at[pl.ds(base, _C)], out_sem[b]).wait()

        for b in range(_NBUF - 1):
            start_gather(b, b)
        start_scatter(_NBUF - 1, _NBUF - 1)

        def visit(j, b):
            b1 = (b - 1) % _NBUF
            wait_scatter(b1)
            start_gather(j + _NBUF - 1, b1)
            wait_gather(b)
            start_scatter(j, b)

        def body(g, carry):
            for b in range(_NBUF):
                visit(g * _NBUF + b, b)
            return carry

        lax.fori_loop(0, n_groups - 1, body, 0)

        g0 = (n_groups - 1) * _NBUF
        for b in range(_NBUF):
            b1 = (b - 1) % _NBUF
            if b == 0:
                wait_scatter(b1)
                start_gather(g0 + _NBUF - 1, b1)
            wait_gather(b)
            start_scatter(g0 + b, b)
        for b in range(_NBUF):
            wait_scatter(b)

    return k(ids3, tableP)


def kernel(input_ids, table):
    b, l = input_ids.shape
    n = b * l
    v, d = table.shape
    assert n % (_NW * _C * _NBUF) == 0
    n_per_w = n // _NW
    n_chunks = n_per_w // _C
    ids3 = input_ids.reshape(_NW, n_chunks, _C)
    tableT = table.T
    rem = v % _C
    tail = tableT[:, v - rem:]
    tableP = _sc_transpose(tableT, tail)
    outP = _sc_gather(ids3, tableP, n_per_w, n_chunks)
    out = lax.slice(outP, (0, 0), (n, d))
    return out.reshape(b, l, d)


# speedup vs baseline: 3.7274x; 1.4612x over previous
import functools

import jax
import jax.numpy as jnp
from jax import lax
from jax.experimental import pallas as pl
from jax.experimental.pallas import tpu as pltpu
from jax.experimental.pallas import tpu_sc as plsc

_NC = 2
_NS = 16
_NW = _NC * _NS
_C = 128
_NBUF = 4
_L = 16


def _make_transpose(rows16):
    diag = [jnp.bitwise_and(rows16 + k, _L - 1) for k in range(_L)]

    def transpose(s, t, nr, nc):
        ncb = nc // _L
        shift = ncb.bit_length() - 1

        @plsc.parallel_loop(0, (nr // _L) * ncb)
        def _(g):
            rb = lax.shift_right_logical(g, shift)
            cb = lax.bitwise_and(g, ncb - 1)
            rbase = rb * _L + rows16
            cbase = cb * _L
            for k in range(_L):
                dk = cbase + diag[k]
                vals = plsc.load_gather(s, [rbase, dk])
                plsc.store_scatter(t, [dk, rbase], vals)

    return transpose


@jax.jit
def _sc_transpose(tableT, tail):
    d, v = tableT.shape
    slabs = v // _C
    rem = v - slabs * _C
    n_visits = -(-slabs // _NW)
    mesh = plsc.VectorSubcoreMesh(core_axis_name="c", subcore_axis_name="s")

    scratch = (
        [pltpu.VMEM((d, _C), jnp.float32) for _ in range(2)]
        + [pltpu.VMEM((_C, _C), jnp.float32) for _ in range(2)]
        + [pltpu.VMEM((d, rem), jnp.float32)]
        + [pltpu.SemaphoreType.DMA for _ in range(4)]
    )

    @functools.partial(
        pl.kernel,
        mesh=mesh,
        out_type=jax.ShapeDtypeStruct((v, _C), jnp.float32),
        scratch_types=scratch,
        compiler_params=pltpu.CompilerParams(
            use_tc_tiling_on_sc=True, needs_layout_passes=False),
    )
    def k(tableT_hbm, tail_hbm, out_hbm, src0, src1, dst0, dst1, tailv,
          in0, in1, out0, out1):
        src = (src0, src1)
        dst = (dst0, dst1)
        in_sem = (in0, in1)
        out_sem = (out0, out1)
        wid = lax.axis_index("s") * _NC + lax.axis_index("c")
        rows16 = lax.iota(jnp.int32, _L)
        transpose = _make_transpose(rows16)

        def c_eff(kv):
            c = wid + kv * _NW
            return jnp.where(c < slabs, c, 0)

        def start_in(kv, b):
            pltpu.async_copy(
                tableT_hbm.at[:, pl.ds(c_eff(kv) * _C, _C)], src[b], in_sem[b])

        def wait_in(b):
            pltpu.make_async_copy(
                tableT_hbm.at[:, pl.ds(0, _C)], src[b], in_sem[b]).wait()

        def start_out(kv, b):
            pltpu.async_copy(
                dst[b], out_hbm.at[pl.ds(c_eff(kv) * _C, _C)], out_sem[b])

        def wait_out(b):
            pltpu.make_async_copy(
                dst[b], out_hbm.at[pl.ds(0, _C)], out_sem[b]).wait()

        start_in(0, 0)
        start_in(1, 1)
        start_out(0, 0)
        start_out(1, 1)

        def visit(kv, b):
            wait_in(b)
            wait_out(b)
            transpose(src[b], dst[b], d, _C)
            start_out(kv, b)
            start_in(kv + 2, b)

        def body(g, carry):
            visit(2 * g, 0)
            visit(2 * g + 1, 1)
            return carry

        lax.fori_loop(0, n_visits // 2, body, 0)

        kv = n_visits - 1
        wait_in(0)
        wait_out(0)
        transpose(src[0], dst[0], d, _C)
        start_out(kv, 0)
        wait_in(1)
        wait_out(1)
        wait_out(0)

        @pl.when(wid == 0)
        def _():
            pltpu.sync_copy(tail_hbm, tailv)
            transpose(tailv, dst0, d, rem)
            pltpu.sync_copy(dst0.at[pl.ds(0, rem), :],
                            out_hbm.at[pl.ds(slabs * _C, rem)])

    return k(tableT, tail)


@functools.partial(jax.jit, static_argnums=(2, 3))
def _sc_gather(ids4, tableP, n_chunks, batch):
    d = 64
    mesh = plsc.VectorSubcoreMesh(core_axis_name="c", subcore_axis_name="s")

    scratch = (
        [pltpu.VMEM((n_chunks, _C), jnp.int32)]
        + [pltpu.VMEM((_C, _C), jnp.float32) for _ in range(_NBUF)]
        + [pltpu.VMEM((1, d, _C), jnp.float32) for _ in range(2)]
        + [pltpu.SemaphoreType.DMA for _ in range(_NBUF + 2)]
    )

    @functools.partial(
        pl.kernel,
        mesh=mesh,
        out_type=jax.ShapeDtypeStruct((n_chunks, d, batch), jnp.float32),
        scratch_types=scratch,
        compiler_params=pltpu.CompilerParams(
            use_tc_tiling_on_sc=True, needs_layout_passes=False),
    )
    def k(ids_hbm, table_hbm, out_hbm, idx_v, *rest):
        gbuf = rest[:_NBUF]
        tbuf = rest[_NBUF:_NBUF + 2]
        in_sem = rest[_NBUF + 2:2 * _NBUF + 2]
        out_sem = rest[2 * _NBUF + 2:]
        wid = lax.axis_index("s") * _NC + lax.axis_index("c")
        b0 = wid * _C
        rows16 = lax.iota(jnp.int32, _L)
        transpose = _make_transpose(rows16)
        pltpu.sync_copy(ids_hbm.at[wid], idx_v)

        def start_gather(jn, b):
            pltpu.async_copy(table_hbm.at[idx_v.at[jn]], gbuf[b], in_sem[b])

        def wait_gather(b):
            pltpu.make_async_copy(
                table_hbm.at[idx_v.at[0]], gbuf[b], in_sem[b]).wait()

        def start_out(j, tb):
            pltpu.async_copy(
                tbuf[tb], out_hbm.at[pl.ds(j, 1), :, pl.ds(b0, _C)],
                out_sem[tb])

        def wait_out(tb):
            pltpu.make_async_copy(
                tbuf[tb], out_hbm.at[pl.ds(0, 1), :, pl.ds(0, _C)],
                out_sem[tb]).wait()

        for b in range(_NBUF - 1):
            start_gather(b, b)
        start_out(0, 0)
        start_out(1, 1)

        def visit(j, b, arm):
            b1 = (b - 1) % _NBUF
            tb = b % 2
            if arm:
                start_gather(j + _NBUF - 1, b1)
            wait_out(tb)
            wait_gather(b)
            transpose(gbuf[b], tbuf[tb].at[0], _C, d)
            start_out(j, tb)

        def body(g, carry):
            for b in range(_NBUF):
                visit(g * _NBUF + b, b, True)
            return carry

        lax.fori_loop(0, n_chunks // _NBUF - 1, body, 0)

        g0 = n_chunks - _NBUF
        for b in range(_NBUF):
            visit(g0 + b, b, b == 0)
        wait_out(0)
        wait_out(1)

    return k(ids4, tableP)


def kernel(input_ids, table):
    bsz, l = input_ids.shape
    v, d = table.shape
    assert bsz == _NW * _C and l % _NBUF == 0 and d == 64
    tableT = table.T
    rem = v % _C
    tail = tableT[:, v - rem:]
    tableP = _sc_transpose(tableT, tail)
    ids4 = input_ids.reshape(_NW, _C, l).transpose(0, 2, 1)
    outN = _sc_gather(ids4, tableP, l, bsz)
    return outN.transpose(2, 0, 1)
